# baseline (speedup 1.0000x reference)
import functools

import jax
import jax.numpy as jnp
from jax import lax
from jax.experimental import pallas as pl
from jax.experimental.pallas import tpu as pltpu
from jax.experimental.pallas import tpu_sc as plsc

_NC = 2
_NS = 16
_NW = _NC * _NS
_WC = 128


def _vq_body(xT_ref, pre_w_ref, pre_b_ref, emb_full_ref, e_sq_ref,
             emb_ref, post_w_ref, post_b_ref, zT_ref, tok_ref, table_ref,
             embTm2_s, ids_s):
    V = emb_full_ref.shape[0]
    T = xT_ref.shape[1]

    @pl.when(pl.program_id(0) == 0)
    def _prep():
        embTm2_s[...] = -2.0 * emb_full_ref[...].T
        ids_s[...] = lax.broadcasted_iota(
            jnp.int32, (1, V), 1).astype(jnp.float32)
    zT = jnp.dot(pre_w_ref[...], xT_ref[...],
                 preferred_element_type=jnp.float32) + pre_b_ref[...]
    zT_ref[...] = zT
    z = zT.T
    R = 128
    for h in range(T // R):
        zh = z[h * R:(h + 1) * R, :]
        z_sq = jnp.sum(zh * zh, axis=1, keepdims=True)
        m2 = jnp.dot(zh, embTm2_s[...], preferred_element_type=jnp.float32)
        acc_d = (z_sq + e_sq_ref[0:1, 0:_WC]) + m2[:, 0:_WC]
        acc_i = jnp.broadcast_to(ids_s[0:1, 0:_WC], (R, _WC))
        for c in range(1, V // _WC):
            lo = c * _WC
            dc = (z_sq + e_sq_ref[0:1, lo:lo + _WC]) + m2[:, lo:lo + _WC]
            better = dc < acc_d
            acc_d = jnp.where(better, dc, acc_d)
            acc_i = jnp.where(better, ids_s[0:1, lo:lo + _WC], acc_i)
        dmin = jnp.min(acc_d, axis=1, keepdims=True)
        tokf = jnp.min(jnp.where(acc_d == dmin, acc_i, jnp.inf), axis=1,
                       keepdims=True)
        tok_ref[h * R:(h + 1) * R, :] = tokf.astype(jnp.int32)
    e_blk = emb_ref[...]
    rec_blk = lax.dot_general(
        e_blk, post_w_ref[...], (((1,), (1,)), ((), ())),
        preferred_element_type=jnp.float32) + post_b_ref[...]
    pad = jnp.zeros((e_blk.shape[0], 32), jnp.float32)
    table_ref[...] = jnp.concatenate([e_blk, rec_blk, pad], axis=1)


def _make_tc_call(N, C, E, V, T, HW, interpret=False):
    G = N // T
    cpb = HW // T
    vb = V // G
    return pl.pallas_call(
        _vq_body,
        grid=(G,),
        interpret=interpret,
        in_specs=[
            pl.BlockSpec((C, T), lambda i: (i // cpb, i % cpb)),
            pl.BlockSpec((E, C), lambda i: (0, 0)),
            pl.BlockSpec((E, 1), lambda i: (0, 0)),
            pl.BlockSpec((V, E), lambda i: (0, 0)),
            pl.BlockSpec((1, V), lambda i: (0, 0)),
            pl.BlockSpec((vb, E), lambda i: (i, 0)),
            pl.BlockSpec((C, E), lambda i: (0, 0)),
            pl.BlockSpec((1, C), lambda i: (0, 0)),
        ],
        scratch_shapes=[
            pltpu.VMEM((E, V), jnp.float32),
            pltpu.VMEM((1, V), jnp.float32),
        ],
        out_specs=[
            pl.BlockSpec((E, T), lambda i: (i // cpb, i % cpb)),
            pl.BlockSpec((T, 1), lambda i: (i, 0)),
            pl.BlockSpec((vb, 128), lambda i: (i, 0)),
        ],
        out_shape=[
            jax.ShapeDtypeStruct((N // HW * E, HW), jnp.float32),
            jax.ShapeDtypeStruct((N, 1), jnp.int32),
            jax.ShapeDtypeStruct((V, 128), jnp.float32),
        ],
    )


def _make_sc_gather(V, N):
    b_per_w = N // _NW
    chunks = b_per_w // 128
    mesh = plsc.VectorSubcoreMesh(core_axis_name="c", subcore_axis_name="s")

    @functools.partial(
        pl.kernel, mesh=mesh,
        out_type=jax.ShapeDtypeStruct((N, 128), jnp.float32),
        scratch_types=[
            pltpu.VMEM((chunks, 128), jnp.int32),
            pltpu.VMEM((b_per_w, 128), jnp.float32),
            pltpu.SemaphoreType.DMA,
        ],
    )
    def k(table_hbm, idx_hbm, out_hbm, idx_v, rows_v, sem):
        wid = lax.axis_index("s") * _NC + lax.axis_index("c")
        base = wid * chunks
        pltpu.sync_copy(idx_hbm.at[pl.ds(base, chunks)], idx_v)
        copies = []
        for j in range(chunks):
            copies.append(pltpu.async_copy(
                table_hbm.at[idx_v.at[j]],
                rows_v.at[pl.ds(j * 128, 128)], sem))
        for c in copies:
            c.wait()
        pltpu.sync_copy(rows_v, out_hbm.at[pl.ds(wid * b_per_w, b_per_w)])

    return k


def kernel(x, pre_w, pre_b, emb, post_w, post_b):
    B, C, H, W = x.shape
    E = pre_w.shape[0]
    V = emb.shape[0]
    N = B * H * W
    HW = H * W
    T = 1024
    x2d = x.reshape(B * C, HW)
    e_sq = jnp.sum(emb ** 2, axis=1).reshape(1, V)
    zT, tok, table = _make_tc_call(N, C, E, V, T, HW)(
        x2d, pre_w, pre_b.reshape(E, 1), emb, e_sq, emb,
        post_w, post_b.reshape(1, C))
    gathered = _make_sc_gather(V, N)(table, tok.reshape(N // 128, 128))
    zq_flat = gathered[:, :E]
    rec_flat = gathered[:, E:E + C]
    z = zT.reshape(B, E, H, W)
    z_q = zq_flat.reshape(B, H, W, E).transpose(0, 3, 1, 2)
    rec = rec_flat.reshape(B, H, W, C).transpose(0, 3, 1, 2)
    return (z, z_q, rec)


# speedup vs baseline: 1.0114x; 1.0114x over previous
import functools

import jax
import jax.numpy as jnp
from jax import lax
from jax.experimental import pallas as pl
from jax.experimental.pallas import tpu as pltpu
from jax.experimental.pallas import tpu_sc as plsc

_NC = 2
_NS = 16
_NW = _NC * _NS
_WC = 128


def _vq_body(xT_ref, pre_w_ref, pre_b_ref, embTm2_ref, e_sq_ref, ids_ref,
             emb_ref, post_wT_ref, post_b_ref, zT_ref, tok_ref, table_ref):
    V = embTm2_ref.shape[1]
    T = xT_ref.shape[1]
    zT = jnp.dot(pre_w_ref[...], xT_ref[...],
                 preferred_element_type=jnp.float32) + pre_b_ref[...]
    zT_ref[...] = zT
    z = zT.T
    R = 128
    for h in range(T // R):
        zh = z[h * R:(h + 1) * R, :]
        z_sq = jnp.sum(zh * zh, axis=1, keepdims=True)
        m2 = jnp.dot(zh, embTm2_ref[...], preferred_element_type=jnp.float32)
        acc_d = (z_sq + e_sq_ref[0:1, 0:_WC]) + m2[:, 0:_WC]
        acc_i = jnp.broadcast_to(ids_ref[0:1, 0:_WC], (R, _WC))
        for c in range(1, V // _WC):
            lo = c * _WC
            dc = (z_sq + e_sq_ref[0:1, lo:lo + _WC]) + m2[:, lo:lo + _WC]
            better = dc < acc_d
            acc_d = jnp.where(better, dc, acc_d)
            acc_i = jnp.where(better, ids_ref[0:1, lo:lo + _WC], acc_i)
        dmin = jnp.min(acc_d, axis=1, keepdims=True)
        tokf = jnp.min(jnp.where(acc_d == dmin, acc_i, jnp.inf), axis=1,
                       keepdims=True)
        tok_ref[h * R:(h + 1) * R, :] = tokf.astype(jnp.int32)
    e_blk = emb_ref[...]
    rec_blk = jnp.dot(e_blk, post_wT_ref[...],
                      preferred_element_type=jnp.float32) + post_b_ref[...]
    pad = jnp.zeros((e_blk.shape[0], 32), jnp.float32)
    table_ref[...] = jnp.concatenate([e_blk, rec_blk, pad], axis=1)


def _make_tc_call(N, C, E, V, T, HW, interpret=False):
    G = N // T
    cpb = HW // T
    vb = V // G
    return pl.pallas_call(
        _vq_body,
        grid=(G,),
        interpret=interpret,
        in_specs=[
            pl.BlockSpec((C, T), lambda i: (i // cpb, i % cpb)),
            pl.BlockSpec((E, C), lambda i: (0, 0)),
            pl.BlockSpec((E, 1), lambda i: (0, 0)),
            pl.BlockSpec((E, V), lambda i: (0, 0)),
            pl.BlockSpec((1, V), lambda i: (0, 0)),
            pl.BlockSpec((1, V), lambda i: (0, 0)),
            pl.BlockSpec((vb, E), lambda i: (i, 0)),
            pl.BlockSpec((E, C), lambda i: (0, 0)),
            pl.BlockSpec((1, C), lambda i: (0, 0)),
        ],
        out_specs=[
            pl.BlockSpec((E, T), lambda i: (i // cpb, i % cpb)),
            pl.BlockSpec((T, 1), lambda i: (i, 0)),
            pl.BlockSpec((vb, 128), lambda i: (i, 0)),
        ],
        out_shape=[
            jax.ShapeDtypeStruct((N // HW * E, HW), jnp.float32),
            jax.ShapeDtypeStruct((N, 1), jnp.int32),
            jax.ShapeDtypeStruct((V, 128), jnp.float32),
        ],
    )


def _make_sc_gather(V, N):
    b_per_w = N // _NW
    chunks = b_per_w // 128
    mesh = plsc.VectorSubcoreMesh(core_axis_name="c", subcore_axis_name="s")

    @functools.partial(
        pl.kernel, mesh=mesh,
        out_type=jax.ShapeDtypeStruct((N, 128), jnp.float32),
        scratch_types=[
            pltpu.VMEM((chunks, 128), jnp.int32),
            pltpu.VMEM((b_per_w, 128), jnp.float32),
            pltpu.SemaphoreType.DMA,
        ],
    )
    def k(table_hbm, idx_hbm, out_hbm, idx_v, rows_v, sem):
        wid = lax.axis_index("s") * _NC + lax.axis_index("c")
        base = wid * chunks
        pltpu.sync_copy(idx_hbm.at[pl.ds(base, chunks)], idx_v)
        copies = []
        for j in range(chunks):
            copies.append(pltpu.async_copy(
                table_hbm.at[idx_v.at[j]],
                rows_v.at[pl.ds(j * 128, 128)], sem))
        for c in copies:
            c.wait()
        pltpu.sync_copy(rows_v, out_hbm.at[pl.ds(wid * b_per_w, b_per_w)])

    return k


def kernel(x, pre_w, pre_b, emb, post_w, post_b):
    B, C, H, W = x.shape
    E = pre_w.shape[0]
    V = emb.shape[0]
    N = B * H * W
    HW = H * W
    T = 1024
    x2d = x.reshape(B * C, HW)
    e_sq = jnp.sum(emb ** 2, axis=1).reshape(1, V)
    ids = jnp.arange(V, dtype=jnp.float32).reshape(1, V)
    zT, tok, table = _make_tc_call(N, C, E, V, T, HW)(
        x2d, pre_w, pre_b.reshape(E, 1), -2.0 * emb.T, e_sq, ids, emb,
        post_w.T, post_b.reshape(1, C))
    gathered = _make_sc_gather(V, N)(table, tok.reshape(N // 128, 128))
    zq_flat = gathered[:, :E]
    rec_flat = gathered[:, E:E + C]
    z = zT.reshape(B, E, H, W)
    z_q = zq_flat.reshape(B, H, W, E).transpose(0, 3, 1, 2)
    rec = rec_flat.reshape(B, H, W, C).transpose(0, 3, 1, 2)
    return (z, z_q, rec)
